# baseline (speedup 1.0000x reference)
import functools

import jax
import jax.numpy as jnp
from jax import lax
from jax.experimental import pallas as pl
from jax.experimental.pallas import tpu as pltpu
from jax.experimental.pallas import tpu_sc as plsc

N_NODES = 50000
N_EDGES = 200000
N_GRAPHS = 2048

NC = 2
NS = 16
NWORK = NC * NS
CHUNK = 128

CE = 49
EW = CE * CHUNK
E_PAD = NWORK * EW
CT = E_PAD // (NS * CHUNK)
ET = CT * CHUNK

HALF = 25600
DUMP_LOCAL = HALF
ACC_ROWS = 25616
T_ROWS = ACC_ROWS // NS
NODE_DUMP = N_NODES

CP = 13
PW = CP * CHUNK
N_PAD2 = NWORK * PW
GRAPH_DUMP = N_GRAPHS
GA_TILE = 136
GA = NS * GA_TILE


def _passes(total, pc):
    return [(i, min(pc, total - i)) for i in range(0, total, pc)]


def _sc_gather(table, idx3d, feat, pc):
    passes = _passes(CE, pc)
    mesh = plsc.VectorSubcoreMesh(core_axis_name="c", subcore_axis_name="s")

    @functools.partial(
        pl.kernel,
        out_type=jax.ShapeDtypeStruct((E_PAD, feat), jnp.float32),
        mesh=mesh,
        scratch_types=[
            pltpu.VMEM((pc, CHUNK), jnp.int32),
            pltpu.VMEM((pc * CHUNK, feat), jnp.float32),
            pltpu.SemaphoreType.DMA,
        ],
        compiler_params=pltpu.CompilerParams(use_tc_tiling_on_sc=False),
    )
    def k(table_hbm, idx_hbm, out_hbm, idxp, buf, sem):
        c = lax.axis_index("c")
        s = lax.axis_index("s")
        wid = c * NS + s
        base = wid * EW
        for p0, cnt in passes:
            pltpu.sync_copy(
                idx_hbm.at[wid, pl.ds(p0, cnt)], idxp.at[pl.ds(0, cnt)]
            )

            def fire(j, carry):
                pltpu.async_copy(
                    table_hbm.at[idxp.at[j]],
                    buf.at[pl.ds(j * CHUNK, CHUNK)],
                    sem,
                )
                return carry

            lax.fori_loop(0, cnt, fire, 0)
            pltpu.make_async_copy(
                table_hbm.at[pl.ds(0, cnt * CHUNK)],
                buf.at[pl.ds(0, cnt * CHUNK)],
                sem,
            ).wait()
            pltpu.sync_copy(
                buf.at[pl.ds(0, cnt * CHUNK)],
                out_hbm.at[pl.ds(base + p0 * CHUNK, cnt * CHUNK)],
            )

    return k(table, idx3d)


def _sc_scatter_nodes(vals, idx4d, pc):
    feat = 16
    nrounds = vals.shape[0]
    passes = _passes(CT, pc)
    buf_rows = pc * CHUNK
    mesh = plsc.VectorSubcoreMesh(core_axis_name="c", subcore_axis_name="s")

    @functools.partial(
        pl.kernel,
        out_type=jax.ShapeDtypeStruct((nrounds, NC, ACC_ROWS, feat), jnp.float32),
        mesh=mesh,
        scratch_types=[
            pltpu.VMEM((pc, CHUNK), jnp.int32),
            pltpu.VMEM((buf_rows, feat), jnp.float32),
            pltpu.VMEM_SHARED((ACC_ROWS, feat), jnp.float32),
        ],
        compiler_params=pltpu.CompilerParams(use_tc_tiling_on_sc=False),
    )
    def k(vals_hbm, idx_hbm, out_hbm, idxp, buf, acc):
        c = lax.axis_index("c")
        s = lax.axis_index("s")
        base = s * ET
        z = jnp.zeros((16,), jnp.float32)

        def zero_row(r, carry):
            buf[r, pl.ds(0, 16)] = z
            return carry

        def one_round(rd, carry):
            lax.fori_loop(0, buf_rows, zero_row, 0)
            for z0, zcnt in _passes(T_ROWS, buf_rows):
                pltpu.sync_copy(
                    buf.at[pl.ds(0, zcnt)],
                    acc.at[pl.ds(s * T_ROWS + z0, zcnt)],
                )
            plsc.subcore_barrier()
            for p0, cnt in passes:
                pltpu.sync_copy(
                    idx_hbm.at[c, s, pl.ds(p0, cnt)], idxp.at[pl.ds(0, cnt)]
                )
                pltpu.sync_copy(
                    vals_hbm.at[rd, pl.ds(base + p0 * CHUNK, cnt * CHUNK)],
                    buf.at[pl.ds(0, cnt * CHUNK)],
                )

                def scat(j, carry2):
                    pltpu.sync_copy(
                        buf.at[pl.ds(j * CHUNK, CHUNK)],
                        acc.at[idxp.at[j]],
                        add=True,
                    )
                    return carry2

                lax.fori_loop(0, cnt, scat, 0)
            plsc.subcore_barrier()
            for z0, zcnt in _passes(T_ROWS, buf_rows):
                pltpu.sync_copy(
                    acc.at[pl.ds(s * T_ROWS + z0, zcnt)],
                    buf.at[pl.ds(0, zcnt)],
                )
                pltpu.sync_copy(
                    buf.at[pl.ds(0, zcnt)],
                    out_hbm.at[rd, c, pl.ds(s * T_ROWS + z0, zcnt)],
                )
            return carry

        lax.fori_loop(0, nrounds, one_round, 0)

    return k(vals, idx4d)


def _sc_scatter_pool(vals, idx3d, pc):
    feat = 16
    passes = _passes(CP, pc)
    buf_rows = pc * CHUNK
    mesh = plsc.VectorSubcoreMesh(core_axis_name="c", subcore_axis_name="s")

    @functools.partial(
        pl.kernel,
        out_type=jax.ShapeDtypeStruct((NC, GA, feat), jnp.float32),
        mesh=mesh,
        scratch_types=[
            pltpu.VMEM((pc, CHUNK), jnp.int32),
            pltpu.VMEM((buf_rows, feat), jnp.float32),
            pltpu.VMEM_SHARED((GA, feat), jnp.float32),
        ],
        compiler_params=pltpu.CompilerParams(use_tc_tiling_on_sc=False),
    )
    def k(vals_hbm, idx_hbm, out_hbm, idxp, buf, acc):
        c = lax.axis_index("c")
        s = lax.axis_index("s")
        wid = c * NS + s
        base = wid * PW
        z = jnp.zeros((16,), jnp.float32)

        def zero_row(r, carry):
            buf[r, pl.ds(0, 16)] = z
            return carry

        lax.fori_loop(0, GA_TILE, zero_row, 0)
        pltpu.sync_copy(
            buf.at[pl.ds(0, GA_TILE)],
            acc.at[pl.ds(s * GA_TILE, GA_TILE)],
        )
        plsc.subcore_barrier()
        for p0, cnt in passes:
            pltpu.sync_copy(
                idx_hbm.at[wid, pl.ds(p0, cnt)], idxp.at[pl.ds(0, cnt)]
            )
            pltpu.sync_copy(
                vals_hbm.at[pl.ds(base + p0 * CHUNK, cnt * CHUNK)],
                buf.at[pl.ds(0, cnt * CHUNK)],
            )

            def scat(j, carry):
                pltpu.sync_copy(
                    buf.at[pl.ds(j * CHUNK, CHUNK)],
                    acc.at[idxp.at[j]],
                    add=True,
                )
                return carry

            lax.fori_loop(0, cnt, scat, 0)
        plsc.subcore_barrier()
        pltpu.sync_copy(
            acc.at[pl.ds(s * GA_TILE, GA_TILE)],
            buf.at[pl.ds(0, GA_TILE)],
        )
        pltpu.sync_copy(
            buf.at[pl.ds(0, GA_TILE)],
            out_hbm.at[c, pl.ds(s * GA_TILE, GA_TILE)],
        )

    return k(vals, idx3d)


def _edge_messages(ea_pad, feats, w1, b1, w2, b2, fin, fout):
    be = 1024
    grid = E_PAD // be
    nsplit = fout // 16

    def body(ea_ref, xf_ref, w1_ref, b1_ref, w2_ref, b2_ref, out_ref):
        h = jnp.maximum(
            jnp.dot(ea_ref[...], w1_ref[...], preferred_element_type=jnp.float32)
            + b1_ref[...],
            0.0,
        )
        wf = (
            jnp.dot(h, w2_ref[...], preferred_element_type=jnp.float32)
            + b2_ref[...]
        )
        xf = xf_ref[...]
        acc = xf[:, 0:1] * wf[:, 0:fout]
        for i in range(1, fin):
            acc = acc + xf[:, i : i + 1] * wf[:, i * fout : (i + 1) * fout]
        for t in range(nsplit):
            out_ref[t] = acc[:, t * 16 : (t + 1) * 16]

    return pl.pallas_call(
        body,
        grid=(grid,),
        in_specs=[
            pl.BlockSpec((be, 4), lambda i: (i, 0)),
            pl.BlockSpec((be, fin), lambda i: (i, 0)),
            pl.BlockSpec((4, 32), lambda i: (0, 0)),
            pl.BlockSpec((1, 32), lambda i: (0, 0)),
            pl.BlockSpec((32, fin * fout), lambda i: (0, 0)),
            pl.BlockSpec((1, fin * fout), lambda i: (0, 0)),
        ],
        out_specs=pl.BlockSpec((nsplit, be, 16), lambda i: (0, i, 0)),
        out_shape=jax.ShapeDtypeStruct((nsplit, E_PAD, 16), jnp.float32),
    )(ea_pad, feats, w1, b1.reshape(1, -1), w2, b2.reshape(1, -1))


def _node_update(agg_list, feats, root, bias, fin, fout):
    bn = 1000
    grid = N_NODES // bn
    nslab = len(agg_list)

    def body(*refs):
        arefs = refs[:nslab]
        x_ref, r_ref, bias_ref, out_ref = refs[nslab:]
        slabs = [a[...] for a in arefs]
        agg = slabs[0] if nslab == 1 else jnp.concatenate(slabs, axis=1)
        out_ref[...] = jnp.maximum(
            agg
            + jnp.dot(x_ref[...], r_ref[...], preferred_element_type=jnp.float32)
            + bias_ref[...],
            0.0,
        )

    return pl.pallas_call(
        body,
        grid=(grid,),
        in_specs=(
            [pl.BlockSpec((bn, 16), lambda i: (i, 0))] * nslab
            + [
                pl.BlockSpec((bn, fin), lambda i: (i, 0)),
                pl.BlockSpec((fin, fout), lambda i: (0, 0)),
                pl.BlockSpec((1, fout), lambda i: (0, 0)),
            ]
        ),
        out_specs=pl.BlockSpec((bn, fout), lambda i: (i, 0)),
        out_shape=jax.ShapeDtypeStruct((N_NODES, fout), jnp.float32),
    )(*agg_list, feats, root, bias.reshape(1, -1))


def _head(pool0, pool1, fc1_w, fc1_b, out_w, out_b):
    def body(p0_ref, p1_ref, w1_ref, b1_ref, w2_ref, b2_ref, out_ref):
        p = p0_ref[...] + p1_ref[...]
        g = jnp.maximum(
            jnp.dot(p, w1_ref[...], preferred_element_type=jnp.float32)
            + b1_ref[...],
            0.0,
        )
        out_ref[...] = (
            jnp.dot(g, w2_ref[...], preferred_element_type=jnp.float32)
            + b2_ref[...]
        )

    return pl.pallas_call(
        body,
        out_shape=jax.ShapeDtypeStruct((N_GRAPHS, 1), jnp.float32),
    )(pool0, pool1, fc1_w, fc1_b.reshape(1, -1), out_w, out_b.reshape(1, -1))


def _concat_halves(agg):
    return jnp.concatenate([agg[0, :HALF], agg[1, :HALF]], axis=0)


def kernel(x, edge_index, edge_attr, batch,
           n1w1, n1b1, n1w2, n1b2, root1, bias1,
           n2w1, n2b1, n2w2, n2b2, root2, bias2,
           fc1_w, fc1_b, out_w, out_b):
    src = edge_index[0].astype(jnp.int32)
    dst = edge_index[1].astype(jnp.int32)
    src3d = jnp.pad(src, (0, E_PAD - N_EDGES)).reshape(NWORK, CE, CHUNK)
    dst_pad = jnp.pad(dst, (0, E_PAD - N_EDGES), constant_values=NODE_DUMP)
    locs = []
    for c in range(NC):
        v = dst_pad - c * HALF
        ok = (v >= 0) & (v < HALF)
        locs.append(jnp.where(ok, v, DUMP_LOCAL))
    dst4d = jnp.stack(locs).reshape(NC, NS, CT, CHUNK)
    ea_pad = jnp.pad(edge_attr, ((0, E_PAD - N_EDGES), (0, 0)))

    xs = _sc_gather(x, src3d, 16, 7)
    msg1 = _edge_messages(ea_pad, xs, n1w1, n1b1, n1w2, n1b2, 16, 32)
    agg1 = _sc_scatter_nodes(msg1, dst4d, 7)
    h1 = _node_update(
        [_concat_halves(agg1[0]), _concat_halves(agg1[1])],
        x, root1, bias1, 16, 32,
    )

    hs = _sc_gather(h1, src3d, 32, 2)
    msg2 = _edge_messages(ea_pad, hs, n2w1, n2b1, n2w2, n2b2, 32, 16)
    agg2 = _sc_scatter_nodes(msg2, dst4d, 7)
    h2 = _node_update([_concat_halves(agg2[0])], h1, root2, bias2, 32, 16)

    h2p = jnp.pad(h2, ((0, N_PAD2 - N_NODES), (0, 0)))
    b3d = jnp.pad(
        batch.astype(jnp.int32), (0, N_PAD2 - N_NODES),
        constant_values=GRAPH_DUMP,
    ).reshape(NWORK, CP, CHUNK)
    poolp = _sc_scatter_pool(h2p, b3d, 4)
    return _head(
        poolp[0, :N_GRAPHS], poolp[1, :N_GRAPHS],
        fc1_w, fc1_b, out_w, out_b,
    )


# speedup vs baseline: 1.1474x; 1.1474x over previous
import functools

import jax
import jax.numpy as jnp
from jax import lax
from jax.experimental import pallas as pl
from jax.experimental.pallas import tpu as pltpu
from jax.experimental.pallas import tpu_sc as plsc

N_NODES = 50000
N_EDGES = 200000
N_GRAPHS = 2048

NC = 2
NS = 16
NWORK = NC * NS
CHUNK = 128

CE = 49
EW = CE * CHUNK
E_PAD = NWORK * EW
CT = E_PAD // (NS * CHUNK)
ET = CT * CHUNK

HALF = 25600
DUMP_LOCAL = HALF
ACC_ROWS = 25616
T_ROWS = ACC_ROWS // NS
NODE_DUMP = N_NODES

CP = 13
PW = CP * CHUNK
N_PAD2 = NWORK * PW
GRAPH_DUMP = N_GRAPHS
GA_TILE = 136
GA = NS * GA_TILE


def _passes(total, pc):
    return [(i, min(pc, total - i)) for i in range(0, total, pc)]


def _sc_gather(table, idx3d, feat, pc):
    passes = _passes(CE, pc)
    mesh = plsc.VectorSubcoreMesh(core_axis_name="c", subcore_axis_name="s")

    @functools.partial(
        pl.kernel,
        out_type=jax.ShapeDtypeStruct((E_PAD, feat), jnp.float32),
        mesh=mesh,
        scratch_types=[
            pltpu.VMEM((pc, CHUNK), jnp.int32),
            pltpu.VMEM((pc * CHUNK, feat), jnp.float32),
            pltpu.SemaphoreType.DMA,
        ],
        compiler_params=pltpu.CompilerParams(use_tc_tiling_on_sc=False),
    )
    def k(table_hbm, idx_hbm, out_hbm, idxp, buf, sem):
        c = lax.axis_index("c")
        s = lax.axis_index("s")
        wid = c * NS + s
        base = wid * EW
        for p0, cnt in passes:
            pltpu.sync_copy(
                idx_hbm.at[wid, pl.ds(p0, cnt)], idxp.at[pl.ds(0, cnt)]
            )

            def fire(j, carry):
                pltpu.async_copy(
                    table_hbm.at[idxp.at[j]],
                    buf.at[pl.ds(j * CHUNK, CHUNK)],
                    sem,
                )
                return carry

            lax.fori_loop(0, cnt, fire, 0)
            pltpu.make_async_copy(
                table_hbm.at[pl.ds(0, cnt * CHUNK)],
                buf.at[pl.ds(0, cnt * CHUNK)],
                sem,
            ).wait()
            pltpu.sync_copy(
                buf.at[pl.ds(0, cnt * CHUNK)],
                out_hbm.at[pl.ds(base + p0 * CHUNK, cnt * CHUNK)],
            )

    return k(table, idx3d)


def _sc_scatter_nodes(vals, idx4d, pc):
    feat = 16
    nrounds = vals.shape[0]
    passes = _passes(CT, pc)
    buf_rows = pc * CHUNK
    mesh = plsc.VectorSubcoreMesh(core_axis_name="c", subcore_axis_name="s")

    @functools.partial(
        pl.kernel,
        out_type=jax.ShapeDtypeStruct((nrounds, NC, ACC_ROWS, feat), jnp.float32),
        mesh=mesh,
        scratch_types=[
            pltpu.VMEM((pc, CHUNK), jnp.int32),
            pltpu.VMEM((buf_rows, feat), jnp.float32),
            pltpu.VMEM_SHARED((ACC_ROWS, feat), jnp.float32),
        ],
        compiler_params=pltpu.CompilerParams(use_tc_tiling_on_sc=False),
    )
    def k(vals_hbm, idx_hbm, out_hbm, idxp, buf, acc):
        c = lax.axis_index("c")
        s = lax.axis_index("s")
        base = s * ET
        z = jnp.zeros((16,), jnp.float32)

        def zero_row(r, carry):
            buf[r, pl.ds(0, 16)] = z
            return carry

        def one_round(rd, carry):
            lax.fori_loop(0, buf_rows, zero_row, 0)
            for z0, zcnt in _passes(T_ROWS, buf_rows):
                pltpu.sync_copy(
                    buf.at[pl.ds(0, zcnt)],
                    acc.at[pl.ds(s * T_ROWS + z0, zcnt)],
                )
            plsc.subcore_barrier()
            for p0, cnt in passes:
                pltpu.sync_copy(
                    idx_hbm.at[c, s, pl.ds(p0, cnt)], idxp.at[pl.ds(0, cnt)]
                )
                pltpu.sync_copy(
                    vals_hbm.at[rd, pl.ds(base + p0 * CHUNK, cnt * CHUNK)],
                    buf.at[pl.ds(0, cnt * CHUNK)],
                )

                def scat(j, carry2):
                    pltpu.sync_copy(
                        buf.at[pl.ds(j * CHUNK, CHUNK)],
                        acc.at[idxp.at[j]],
                        add=True,
                    )
                    return carry2

                lax.fori_loop(0, cnt, scat, 0)
            plsc.subcore_barrier()
            for z0, zcnt in _passes(T_ROWS, buf_rows):
                pltpu.sync_copy(
                    acc.at[pl.ds(s * T_ROWS + z0, zcnt)],
                    buf.at[pl.ds(0, zcnt)],
                )
                pltpu.sync_copy(
                    buf.at[pl.ds(0, zcnt)],
                    out_hbm.at[rd, c, pl.ds(s * T_ROWS + z0, zcnt)],
                )
            return carry

        lax.fori_loop(0, nrounds, one_round, 0)

    return k(vals, idx4d)


def _sc_scatter_pool(vals, idx3d, pc):
    feat = 16
    passes = _passes(CP, pc)
    buf_rows = pc * CHUNK
    mesh = plsc.VectorSubcoreMesh(core_axis_name="c", subcore_axis_name="s")

    @functools.partial(
        pl.kernel,
        out_type=jax.ShapeDtypeStruct((NC, GA, feat), jnp.float32),
        mesh=mesh,
        scratch_types=[
            pltpu.VMEM((pc, CHUNK), jnp.int32),
            pltpu.VMEM((buf_rows, feat), jnp.float32),
            pltpu.VMEM_SHARED((GA, feat), jnp.float32),
        ],
        compiler_params=pltpu.CompilerParams(use_tc_tiling_on_sc=False),
    )
    def k(vals_hbm, idx_hbm, out_hbm, idxp, buf, acc):
        c = lax.axis_index("c")
        s = lax.axis_index("s")
        wid = c * NS + s
        base = wid * PW
        z = jnp.zeros((16,), jnp.float32)

        def zero_row(r, carry):
            buf[r, pl.ds(0, 16)] = z
            return carry

        lax.fori_loop(0, GA_TILE, zero_row, 0)
        pltpu.sync_copy(
            buf.at[pl.ds(0, GA_TILE)],
            acc.at[pl.ds(s * GA_TILE, GA_TILE)],
        )
        plsc.subcore_barrier()
        for p0, cnt in passes:
            pltpu.sync_copy(
                idx_hbm.at[wid, pl.ds(p0, cnt)], idxp.at[pl.ds(0, cnt)]
            )
            pltpu.sync_copy(
                vals_hbm.at[pl.ds(base + p0 * CHUNK, cnt * CHUNK)],
                buf.at[pl.ds(0, cnt * CHUNK)],
            )

            def scat(j, carry):
                pltpu.sync_copy(
                    buf.at[pl.ds(j * CHUNK, CHUNK)],
                    acc.at[idxp.at[j]],
                    add=True,
                )
                return carry

            lax.fori_loop(0, cnt, scat, 0)
        plsc.subcore_barrier()
        pltpu.sync_copy(
            acc.at[pl.ds(s * GA_TILE, GA_TILE)],
            buf.at[pl.ds(0, GA_TILE)],
        )
        pltpu.sync_copy(
            buf.at[pl.ds(0, GA_TILE)],
            out_hbm.at[c, pl.ds(s * GA_TILE, GA_TILE)],
        )

    return k(vals, idx3d)


def _edge_messages(ea_pad, feats, w1, b1, w2, b2, fin, fout):
    be = 1024
    grid = E_PAD // be
    nsplit = fout // 16
    kdim = 32 * fin + fin

    tb = jnp.concatenate(
        [
            w2.reshape(32, fin, fout).transpose(1, 0, 2).reshape(fin * 32, fout),
            b2.reshape(fin, fout),
        ],
        axis=0,
    )

    def body(ea_ref, xf_ref, w1_ref, b1_ref, tb_ref, out_ref):
        h = jnp.maximum(
            jnp.dot(ea_ref[...], w1_ref[...], preferred_element_type=jnp.float32)
            + b1_ref[...],
            0.0,
        )
        xf = xf_ref[...]
        z = jnp.concatenate(
            [xf[:, i : i + 1] * h for i in range(fin)] + [xf], axis=1
        )
        acc = jnp.dot(z, tb_ref[...], preferred_element_type=jnp.float32)
        for t in range(nsplit):
            out_ref[t] = acc[:, t * 16 : (t + 1) * 16]

    return pl.pallas_call(
        body,
        grid=(grid,),
        in_specs=[
            pl.BlockSpec((be, 4), lambda i: (i, 0)),
            pl.BlockSpec((be, fin), lambda i: (i, 0)),
            pl.BlockSpec((4, 32), lambda i: (0, 0)),
            pl.BlockSpec((1, 32), lambda i: (0, 0)),
            pl.BlockSpec((kdim, fout), lambda i: (0, 0)),
        ],
        out_specs=pl.BlockSpec((nsplit, be, 16), lambda i: (0, i, 0)),
        out_shape=jax.ShapeDtypeStruct((nsplit, E_PAD, 16), jnp.float32),
    )(ea_pad, feats, w1, b1.reshape(1, -1), tb)


def _node_update(agg_list, feats, root, bias, fin, fout):
    bn = 1000
    grid = N_NODES // bn
    nslab = len(agg_list)

    def body(*refs):
        arefs = refs[:nslab]
        x_ref, r_ref, bias_ref, out_ref = refs[nslab:]
        slabs = [a[...] for a in arefs]
        agg = slabs[0] if nslab == 1 else jnp.concatenate(slabs, axis=1)
        out_ref[...] = jnp.maximum(
            agg
            + jnp.dot(x_ref[...], r_ref[...], preferred_element_type=jnp.float32)
            + bias_ref[...],
            0.0,
        )

    return pl.pallas_call(
        body,
        grid=(grid,),
        in_specs=(
            [pl.BlockSpec((bn, 16), lambda i: (i, 0))] * nslab
            + [
                pl.BlockSpec((bn, fin), lambda i: (i, 0)),
                pl.BlockSpec((fin, fout), lambda i: (0, 0)),
                pl.BlockSpec((1, fout), lambda i: (0, 0)),
            ]
        ),
        out_specs=pl.BlockSpec((bn, fout), lambda i: (i, 0)),
        out_shape=jax.ShapeDtypeStruct((N_NODES, fout), jnp.float32),
    )(*agg_list, feats, root, bias.reshape(1, -1))


def _head(pool0, pool1, fc1_w, fc1_b, out_w, out_b):
    def body(p0_ref, p1_ref, w1_ref, b1_ref, w2_ref, b2_ref, out_ref):
        p = p0_ref[...] + p1_ref[...]
        g = jnp.maximum(
            jnp.dot(p, w1_ref[...], preferred_element_type=jnp.float32)
            + b1_ref[...],
            0.0,
        )
        out_ref[...] = (
            jnp.dot(g, w2_ref[...], preferred_element_type=jnp.float32)
            + b2_ref[...]
        )

    return pl.pallas_call(
        body,
        out_shape=jax.ShapeDtypeStruct((N_GRAPHS, 1), jnp.float32),
    )(pool0, pool1, fc1_w, fc1_b.reshape(1, -1), out_w, out_b.reshape(1, -1))


def _concat_halves(agg):
    return jnp.concatenate([agg[0, :HALF], agg[1, :HALF]], axis=0)


def kernel(x, edge_index, edge_attr, batch,
           n1w1, n1b1, n1w2, n1b2, root1, bias1,
           n2w1, n2b1, n2w2, n2b2, root2, bias2,
           fc1_w, fc1_b, out_w, out_b):
    src = edge_index[0].astype(jnp.int32)
    dst = edge_index[1].astype(jnp.int32)
    src3d = jnp.pad(src, (0, E_PAD - N_EDGES)).reshape(NWORK, CE, CHUNK)
    dst_pad = jnp.pad(dst, (0, E_PAD - N_EDGES), constant_values=NODE_DUMP)
    locs = []
    for c in range(NC):
        v = dst_pad - c * HALF
        ok = (v >= 0) & (v < HALF)
        locs.append(jnp.where(ok, v, DUMP_LOCAL))
    dst4d = jnp.stack(locs).reshape(NC, NS, CT, CHUNK)
    ea_pad = jnp.pad(edge_attr, ((0, E_PAD - N_EDGES), (0, 0)))

    xs = _sc_gather(x, src3d, 16, 7)
    msg1 = _edge_messages(ea_pad, xs, n1w1, n1b1, n1w2, n1b2, 16, 32)
    agg1 = _sc_scatter_nodes(msg1, dst4d, 7)
    h1 = _node_update(
        [_concat_halves(agg1[0]), _concat_halves(agg1[1])],
        x, root1, bias1, 16, 32,
    )

    hs = _sc_gather(h1, src3d, 32, 2)
    msg2 = _edge_messages(ea_pad, hs, n2w1, n2b1, n2w2, n2b2, 32, 16)
    agg2 = _sc_scatter_nodes(msg2, dst4d, 7)
    h2 = _node_update([_concat_halves(agg2[0])], h1, root2, bias2, 32, 16)

    h2p = jnp.pad(h2, ((0, N_PAD2 - N_NODES), (0, 0)))
    b3d = jnp.pad(
        batch.astype(jnp.int32), (0, N_PAD2 - N_NODES),
        constant_values=GRAPH_DUMP,
    ).reshape(NWORK, CP, CHUNK)
    poolp = _sc_scatter_pool(h2p, b3d, 4)
    return _head(
        poolp[0, :N_GRAPHS], poolp[1, :N_GRAPHS],
        fc1_w, fc1_b, out_w, out_b,
    )


# speedup vs baseline: 2.0385x; 1.7765x over previous
import functools

import jax
import jax.numpy as jnp
from jax import lax
from jax.experimental import pallas as pl
from jax.experimental.pallas import tpu as pltpu
from jax.experimental.pallas import tpu_sc as plsc

N_NODES = 50000
N_EDGES = 200000
N_GRAPHS = 2048

NC = 2
NS = 16
NWORK = NC * NS
CHUNK = 128

CE = 49
EW = CE * CHUNK
E_PAD = NWORK * EW
CT = E_PAD // (NS * CHUNK)
ET = CT * CHUNK

HALF = 25600
DUMP_LOCAL = HALF
ACC_ROWS = 25616
T_ROWS = ACC_ROWS // NS
NODE_DUMP = N_NODES

CP = 13
PW = CP * CHUNK
N_PAD2 = NWORK * PW
GRAPH_DUMP = N_GRAPHS
GA_TILE = 136
GA = NS * GA_TILE


def _passes(total, pc):
    return [(i, min(pc, total - i)) for i in range(0, total, pc)]


def _sc_gather(table, idx3d, feat, pc):
    passes = _passes(CE, pc)
    mesh = plsc.VectorSubcoreMesh(core_axis_name="c", subcore_axis_name="s")

    @functools.partial(
        pl.kernel,
        out_type=jax.ShapeDtypeStruct((E_PAD, feat), jnp.float32),
        mesh=mesh,
        scratch_types=[
            pltpu.VMEM((pc, CHUNK), jnp.int32),
            pltpu.VMEM((pc * CHUNK, feat), jnp.float32),
            pltpu.SemaphoreType.DMA,
        ],
        compiler_params=pltpu.CompilerParams(use_tc_tiling_on_sc=False),
    )
    def k(table_hbm, idx_hbm, out_hbm, idxp, buf, sem):
        c = lax.axis_index("c")
        s = lax.axis_index("s")
        wid = c * NS + s
        base = wid * EW
        for p0, cnt in passes:
            pltpu.sync_copy(
                idx_hbm.at[wid, pl.ds(p0, cnt)], idxp.at[pl.ds(0, cnt)]
            )

            def fire(j, carry):
                pltpu.async_copy(
                    table_hbm.at[idxp.at[j]],
                    buf.at[pl.ds(j * CHUNK, CHUNK)],
                    sem,
                )
                return carry

            lax.fori_loop(0, cnt, fire, 0)
            pltpu.make_async_copy(
                table_hbm.at[pl.ds(0, cnt * CHUNK)],
                buf.at[pl.ds(0, cnt * CHUNK)],
                sem,
            ).wait()
            pltpu.sync_copy(
                buf.at[pl.ds(0, cnt * CHUNK)],
                out_hbm.at[pl.ds(base + p0 * CHUNK, cnt * CHUNK)],
            )

    return k(table, idx3d)


def _sc_scatter_nodes(vals, idx4d, pc):
    feat = 16
    nrounds = vals.shape[0]
    passes = _passes(CT, pc)
    buf_rows = pc * CHUNK
    mesh = plsc.VectorSubcoreMesh(core_axis_name="c", subcore_axis_name="s")

    @functools.partial(
        pl.kernel,
        out_type=jax.ShapeDtypeStruct((nrounds, NC, ACC_ROWS, feat), jnp.float32),
        mesh=mesh,
        scratch_types=[
            pltpu.VMEM((pc, CHUNK), jnp.int32),
            pltpu.VMEM((buf_rows, feat), jnp.float32),
            pltpu.VMEM_SHARED((ACC_ROWS, feat), jnp.float32),
        ],
        compiler_params=pltpu.CompilerParams(use_tc_tiling_on_sc=False),
    )
    def k(vals_hbm, idx_hbm, out_hbm, idxp, buf, acc):
        c = lax.axis_index("c")
        s = lax.axis_index("s")
        base = s * ET
        z = jnp.zeros((16,), jnp.float32)

        def zero_row(r, carry):
            buf[r, pl.ds(0, 16)] = z
            return carry

        def one_round(rd, carry):
            lax.fori_loop(0, buf_rows, zero_row, 0)
            for z0, zcnt in _passes(T_ROWS, buf_rows):
                pltpu.sync_copy(
                    buf.at[pl.ds(0, zcnt)],
                    acc.at[pl.ds(s * T_ROWS + z0, zcnt)],
                )
            plsc.subcore_barrier()
            for p0, cnt in passes:
                pltpu.sync_copy(
                    idx_hbm.at[c, s, pl.ds(p0, cnt)], idxp.at[pl.ds(0, cnt)]
                )
                pltpu.sync_copy(
                    vals_hbm.at[rd, pl.ds(base + p0 * CHUNK, cnt * CHUNK)],
                    buf.at[pl.ds(0, cnt * CHUNK)],
                )

                def scat(j, carry2):
                    pltpu.sync_copy(
                        buf.at[pl.ds(j * CHUNK, CHUNK)],
                        acc.at[idxp.at[j]],
                        add=True,
                    )
                    return carry2

                lax.fori_loop(0, cnt, scat, 0)
            plsc.subcore_barrier()
            for z0, zcnt in _passes(T_ROWS, buf_rows):
                pltpu.sync_copy(
                    acc.at[pl.ds(s * T_ROWS + z0, zcnt)],
                    buf.at[pl.ds(0, zcnt)],
                )
                pltpu.sync_copy(
                    buf.at[pl.ds(0, zcnt)],
                    out_hbm.at[rd, c, pl.ds(s * T_ROWS + z0, zcnt)],
                )
            return carry

        lax.fori_loop(0, nrounds, one_round, 0)

    return k(vals, idx4d)


def _sc_scatter_pool(vals, idx3d, pc):
    feat = 16
    passes = _passes(CP, pc)
    buf_rows = pc * CHUNK
    mesh = plsc.VectorSubcoreMesh(core_axis_name="c", subcore_axis_name="s")

    @functools.partial(
        pl.kernel,
        out_type=jax.ShapeDtypeStruct((NC, GA, feat), jnp.float32),
        mesh=mesh,
        scratch_types=[
            pltpu.VMEM((pc, CHUNK), jnp.int32),
            pltpu.VMEM((buf_rows, feat), jnp.float32),
            pltpu.VMEM_SHARED((GA, feat), jnp.float32),
        ],
        compiler_params=pltpu.CompilerParams(use_tc_tiling_on_sc=False),
    )
    def k(vals_hbm, idx_hbm, out_hbm, idxp, buf, acc):
        c = lax.axis_index("c")
        s = lax.axis_index("s")
        wid = c * NS + s
        base = wid * PW
        z = jnp.zeros((16,), jnp.float32)

        def zero_row(r, carry):
            buf[r, pl.ds(0, 16)] = z
            return carry

        lax.fori_loop(0, GA_TILE, zero_row, 0)
        pltpu.sync_copy(
            buf.at[pl.ds(0, GA_TILE)],
            acc.at[pl.ds(s * GA_TILE, GA_TILE)],
        )
        plsc.subcore_barrier()
        for p0, cnt in passes:
            pltpu.sync_copy(
                idx_hbm.at[wid, pl.ds(p0, cnt)], idxp.at[pl.ds(0, cnt)]
            )
            pltpu.sync_copy(
                vals_hbm.at[pl.ds(base + p0 * CHUNK, cnt * CHUNK)],
                buf.at[pl.ds(0, cnt * CHUNK)],
            )

            def scat(j, carry):
                pltpu.sync_copy(
                    buf.at[pl.ds(j * CHUNK, CHUNK)],
                    acc.at[idxp.at[j]],
                    add=True,
                )
                return carry

            lax.fori_loop(0, cnt, scat, 0)
        plsc.subcore_barrier()
        pltpu.sync_copy(
            acc.at[pl.ds(s * GA_TILE, GA_TILE)],
            buf.at[pl.ds(0, GA_TILE)],
        )
        pltpu.sync_copy(
            buf.at[pl.ds(0, GA_TILE)],
            out_hbm.at[c, pl.ds(s * GA_TILE, GA_TILE)],
        )

    return k(vals, idx3d)


def _edge_messages(ea_pad, feats, w1, b1, w2, b2, fin, fout):
    be = 1024
    grid = E_PAD // be
    nsplit = fout // 16
    kdim = 32 * fin + fin

    tbh = (
        w2.reshape(32, fin, fout)
        .transpose(1, 0, 2)
        .reshape(fin * 32, fout)
        .astype(jnp.bfloat16)
    )
    bmat = b2.reshape(fin, fout)
    e4 = jnp.repeat(jnp.eye(4, dtype=jnp.float32), 32, axis=1).astype(jnp.bfloat16)

    def body(ea_ref, xf_ref, w1_ref, b1_ref, tbh_ref, bm_ref, e4_ref, out_ref):
        h = jnp.maximum(
            jnp.dot(ea_ref[...], w1_ref[...], preferred_element_type=jnp.float32)
            + b1_ref[...],
            0.0,
        ).astype(jnp.bfloat16)
        xf = xf_ref[...]
        xb = xf.astype(jnp.bfloat16)
        hh = jnp.concatenate([h, h, h, h], axis=1)
        acc = jnp.dot(xf, bm_ref[...], preferred_element_type=jnp.float32)
        for g in range(0, fin, 4):
            zg = (
                jnp.dot(
                    xb[:, g : g + 4],
                    e4_ref[...],
                    preferred_element_type=jnp.float32,
                ).astype(jnp.bfloat16)
                * hh
            )
            acc = acc + jnp.dot(
                zg,
                tbh_ref[pl.ds(g * 32, 128), :],
                preferred_element_type=jnp.float32,
            )
        for t in range(nsplit):
            out_ref[t] = acc[:, t * 16 : (t + 1) * 16]

    return pl.pallas_call(
        body,
        grid=(grid,),
        in_specs=[
            pl.BlockSpec((be, 4), lambda i: (i, 0)),
            pl.BlockSpec((be, fin), lambda i: (i, 0)),
            pl.BlockSpec((4, 32), lambda i: (0, 0)),
            pl.BlockSpec((1, 32), lambda i: (0, 0)),
            pl.BlockSpec((fin * 32, fout), lambda i: (0, 0)),
            pl.BlockSpec((fin, fout), lambda i: (0, 0)),
            pl.BlockSpec((4, 128), lambda i: (0, 0)),
        ],
        out_specs=pl.BlockSpec((nsplit, be, 16), lambda i: (0, i, 0)),
        out_shape=jax.ShapeDtypeStruct((nsplit, E_PAD, 16), jnp.float32),
    )(ea_pad, feats, w1, b1.reshape(1, -1), tbh, bmat, e4)


def _node_update(agg_list, feats, root, bias, fin, fout):
    bn = 1000
    grid = N_NODES // bn
    nslab = len(agg_list)

    def body(*refs):
        arefs = refs[:nslab]
        x_ref, r_ref, bias_ref, out_ref = refs[nslab:]
        slabs = [a[...] for a in arefs]
        agg = slabs[0] if nslab == 1 else jnp.concatenate(slabs, axis=1)
        out_ref[...] = jnp.maximum(
            agg
            + jnp.dot(x_ref[...], r_ref[...], preferred_element_type=jnp.float32)
            + bias_ref[...],
            0.0,
        )

    return pl.pallas_call(
        body,
        grid=(grid,),
        in_specs=(
            [pl.BlockSpec((bn, 16), lambda i: (i, 0))] * nslab
            + [
                pl.BlockSpec((bn, fin), lambda i: (i, 0)),
                pl.BlockSpec((fin, fout), lambda i: (0, 0)),
                pl.BlockSpec((1, fout), lambda i: (0, 0)),
            ]
        ),
        out_specs=pl.BlockSpec((bn, fout), lambda i: (i, 0)),
        out_shape=jax.ShapeDtypeStruct((N_NODES, fout), jnp.float32),
    )(*agg_list, feats, root, bias.reshape(1, -1))


def _head(pool0, pool1, fc1_w, fc1_b, out_w, out_b):
    def body(p0_ref, p1_ref, w1_ref, b1_ref, w2_ref, b2_ref, out_ref):
        p = p0_ref[...] + p1_ref[...]
        g = jnp.maximum(
            jnp.dot(p, w1_ref[...], preferred_element_type=jnp.float32)
            + b1_ref[...],
            0.0,
        )
        out_ref[...] = (
            jnp.dot(g, w2_ref[...], preferred_element_type=jnp.float32)
            + b2_ref[...]
        )

    return pl.pallas_call(
        body,
        out_shape=jax.ShapeDtypeStruct((N_GRAPHS, 1), jnp.float32),
    )(pool0, pool1, fc1_w, fc1_b.reshape(1, -1), out_w, out_b.reshape(1, -1))


def _concat_halves(agg):
    return jnp.concatenate([agg[0, :HALF], agg[1, :HALF]], axis=0)


def kernel(x, edge_index, edge_attr, batch,
           n1w1, n1b1, n1w2, n1b2, root1, bias1,
           n2w1, n2b1, n2w2, n2b2, root2, bias2,
           fc1_w, fc1_b, out_w, out_b):
    src = edge_index[0].astype(jnp.int32)
    dst = edge_index[1].astype(jnp.int32)
    src3d = jnp.pad(src, (0, E_PAD - N_EDGES)).reshape(NWORK, CE, CHUNK)
    dst_pad = jnp.pad(dst, (0, E_PAD - N_EDGES), constant_values=NODE_DUMP)
    locs = []
    for c in range(NC):
        v = dst_pad - c * HALF
        ok = (v >= 0) & (v < HALF)
        locs.append(jnp.where(ok, v, DUMP_LOCAL))
    dst4d = jnp.stack(locs).reshape(NC, NS, CT, CHUNK)
    ea_pad = jnp.pad(edge_attr, ((0, E_PAD - N_EDGES), (0, 0)))

    xs = _sc_gather(x, src3d, 16, 7)
    msg1 = _edge_messages(ea_pad, xs, n1w1, n1b1, n1w2, n1b2, 16, 32)
    agg1 = _sc_scatter_nodes(msg1, dst4d, 7)
    h1 = _node_update(
        [_concat_halves(agg1[0]), _concat_halves(agg1[1])],
        x, root1, bias1, 16, 32,
    )

    hs = _sc_gather(h1, src3d, 32, 2)
    msg2 = _edge_messages(ea_pad, hs, n2w1, n2b1, n2w2, n2b2, 32, 16)
    agg2 = _sc_scatter_nodes(msg2, dst4d, 7)
    h2 = _node_update([_concat_halves(agg2[0])], h1, root2, bias2, 32, 16)

    h2p = jnp.pad(h2, ((0, N_PAD2 - N_NODES), (0, 0)))
    b3d = jnp.pad(
        batch.astype(jnp.int32), (0, N_PAD2 - N_NODES),
        constant_values=GRAPH_DUMP,
    ).reshape(NWORK, CP, CHUNK)
    poolp = _sc_scatter_pool(h2p, b3d, 4)
    return _head(
        poolp[0, :N_GRAPHS], poolp[1, :N_GRAPHS],
        fc1_w, fc1_b, out_w, out_b,
    )
